# Initial kernel scaffold; baseline (speedup 1.0000x reference)
import jax
import jax.numpy as jnp
from jax.experimental import pallas as pl


def kernel(x, edge_index, W1, a_src1, a_dst1, b1, W2, a_src2, a_dst2, b2, lw1, lb1, lw2, lb2):
    raise NotImplementedError("write your pallas kernel here")


# speedup vs baseline: 24.8280x; 24.8280x over previous
import functools

import jax
import jax.numpy as jnp
from jax import lax
from jax.experimental import pallas as pl
from jax.experimental.pallas import tpu as pltpu
from jax.experimental.pallas import tpu_sc as plsc

F32 = jnp.float32

_NC = 2
_NS = 16
_NW = _NC * _NS

_HEADS = 32
_C = 16
_G = 2 * _HEADS
_XWW = 2 * _HEADS * _C

_CH1 = 128
_CH2 = 64
_TCR = 1024


def _elu(v):
    return jnp.where(v > 0.0, v, jnp.exp(v) - 1.0)


def _leaky(v):
    return jnp.where(v > 0.0, v, 0.2 * v)


def _tc1_body(x_ref, w12_ref, asf_ref, adf_ref, s2_ref, lw1_ref, lb1_ref,
              lw2_ref, lb2_ref, xw_ref, att_ref, exs_ref, xself_ref):
    xb = x_ref[...]
    w12 = w12_ref[...]
    xw = jnp.dot(xb, w12, preferred_element_type=F32)
    xw_ref[...] = xw
    s2 = s2_ref[...]
    ws = jnp.dot(w12 * asf_ref[...], s2, preferred_element_type=F32)
    wd = jnp.dot(w12 * adf_ref[...], s2, preferred_element_type=F32)
    asv = jnp.dot(xb, ws, preferred_element_type=F32)
    adv = jnp.dot(xb, wd, preferred_element_type=F32)
    att_ref[...] = jnp.concatenate([asv, adv], axis=1)
    exs_ref[...] = jnp.exp(_leaky(asv + adv))
    x1 = _elu(jnp.dot(xb, lw1_ref[...], preferred_element_type=F32)
              + lb1_ref[...])
    xself_ref[...] = _elu(jnp.dot(x1, lw2_ref[...],
                                  preferred_element_type=F32) + lb2_ref[...])


def _tc2_body(den_ref, exs_ref, xw_ref, s2t_ref, sc2_ref,
              rden_ref, selfnum_ref):
    exs = exs_ref[...]
    rd = 1.0 / (den_ref[0] + den_ref[1] + exs)
    rden_ref[...] = jnp.concatenate([rd, rd], axis=1)
    w = exs * rd
    wexp = jnp.dot(w, s2t_ref[...], preferred_element_type=F32)
    selfnum_ref[...] = jnp.dot(xw_ref[...] * wexp, sc2_ref[...],
                               preferred_element_type=F32)


def _tc3_body(num_ref, selfnum_ref, b12_ref, out_ref):
    t = (num_ref[0] + num_ref[1] + selfnum_ref[...]) * (1.0 / _HEADS) \
        + b12_ref[...]
    out_ref[...] = _elu(t)


def _sc1_body(npad, ep_w, atttab, sidx, didx, ex_out, den_out,
              isv, idv, idv2, srows, drows, exv, exw, den_sh, sem1, sem2):
    cid = lax.axis_index("c")
    sid = lax.axis_index("s")
    wid = sid * _NC + cid
    nrows = npad // 2
    rows_per_tile = nrows // _NS
    zero = jnp.zeros((16,), F32)

    def zrow(j, carry):
        for p in range(8):
            exw[j, pl.ds(16 * p, 16)] = zero
        return carry

    lax.fori_loop(0, _CH1, zrow, 0)
    row0 = sid * rows_per_tile
    for b in range(rows_per_tile // 64):
        pltpu.sync_copy(exw.at[pl.ds(0, 64)],
                        den_sh.at[pl.ds(row0 + b * 64, 64)])
    plsc.subcore_barrier()

    base_w = wid * ep_w

    def chunk(k, carry):
        base = base_w + k * _CH1
        pltpu.sync_copy(sidx.at[pl.ds(base, _CH1)], isv)
        pltpu.sync_copy(didx.at[pl.ds(base, _CH1)], idv)
        cp1 = pltpu.async_copy(atttab.at[isv], srows, sem1)
        cp2 = pltpu.async_copy(atttab.at[idv], drows, sem2)
        cp1.wait()
        cp2.wait()

        def egrp(g, c2):
            dvec = idv[pl.ds(g * 16, 16)]
            idv2[pl.ds(g * 16, 16)] = jax.lax.shift_right_logical(dvec, 1)
            colv = (dvec & 1) * _G
            for i in range(16):
                j = g * 16 + i
                colb = colv[i]
                ncolb = _G - colb
                for p in range(4):
                    a = (srows[j, pl.ds(16 * p, 16)]
                         + drows[j, pl.ds(_G + 16 * p, 16)])
                    e = jnp.exp(_leaky(a))
                    exv[j, pl.ds(16 * p, 16)] = e
                    exw[j, pl.ds(colb + 16 * p, 16)] = e
                    exw[j, pl.ds(ncolb + 16 * p, 16)] = zero
            return c2

        lax.fori_loop(0, _CH1 // 16, egrp, 0)
        pltpu.sync_copy(exv, ex_out.at[pl.ds(base, _CH1)])
        pltpu.sync_copy(exw, den_sh.at[idv2], add=True)
        return carry

    lax.fori_loop(0, ep_w // _CH1, chunk, 0)
    plsc.subcore_barrier()
    for b in range(rows_per_tile // 64):
        pltpu.sync_copy(den_sh.at[pl.ds(row0 + b * 64, 64)],
                        exw.at[pl.ds(0, 64)])
        pltpu.sync_copy(exw.at[pl.ds(0, 64)],
                        den_out.at[cid, pl.ds(row0 + b * 64, 64)])


def _sc2_body(npad, ep_w, xwtab, rdentab, exbuf, sidx, didx, num_out,
              isv, idv, idv2, xwrows, rdrows, exv, msgv, num_sh, sem1, sem2):
    cid = lax.axis_index("c")
    sid = lax.axis_index("s")
    wid = sid * _NC + cid
    nrows = npad // 4
    rows_per_tile = nrows // _NS
    zero = jnp.zeros((16,), F32)

    def zrow(j, carry):
        for p in range(8):
            msgv[j, pl.ds(16 * p, 16)] = zero
        return carry

    lax.fori_loop(0, _CH2, zrow, 0)
    row0 = sid * rows_per_tile
    for b in range(rows_per_tile // 32):
        pltpu.sync_copy(msgv.at[pl.ds(0, 32)],
                        num_sh.at[pl.ds(row0 + b * 32, 32)])
    plsc.subcore_barrier()

    base_w = wid * ep_w

    def chunk(k, carry):
        base = base_w + k * _CH2
        pltpu.sync_copy(sidx.at[pl.ds(base, _CH2)], isv)
        pltpu.sync_copy(didx.at[pl.ds(base, _CH2)], idv)
        cp1 = pltpu.async_copy(xwtab.at[isv], xwrows, sem1)
        cp2 = pltpu.async_copy(rdentab.at[idv], rdrows, sem2)
        pltpu.sync_copy(exbuf.at[pl.ds(base, _CH2)], exv)
        cp1.wait()
        cp2.wait()

        def egrp(g, c2):
            dvec = idv[pl.ds(g * 16, 16)]
            idv2[pl.ds(g * 16, 16)] = jax.lax.shift_right_logical(dvec, 2)
            colv = (dvec & 3) * 32
            for i in range(16):
                j = g * 16 + i
                colb = colv[i]
                wv = [exv[j, pl.ds(16 * p, 16)] * rdrows[j, pl.ds(16 * p, 16)]
                      for p in range(4)]
                acc1 = jnp.zeros((16,), F32)
                acc2 = jnp.zeros((16,), F32)
                for h in range(_HEADS):
                    acc1 = acc1 + wv[h // 16][h % 16] \
                        * xwrows[j, pl.ds(h * 16, 16)]
                    acc2 = acc2 + wv[2 + h // 16][h % 16] \
                        * xwrows[j, pl.ds(512 + h * 16, 16)]
                msgv[j, pl.ds(colb, 16)] = acc1
                msgv[j, pl.ds(colb + 16, 16)] = acc2
                for q in (32, 64, 96):
                    off = (colb + q) & 127
                    msgv[j, pl.ds(off, 16)] = zero
                    msgv[j, pl.ds(off + 16, 16)] = zero
            return c2

        lax.fori_loop(0, _CH2 // 16, egrp, 0)
        pltpu.sync_copy(msgv, num_sh.at[idv2], add=True)
        return carry

    lax.fori_loop(0, ep_w // _CH2, chunk, 0)
    plsc.subcore_barrier()
    for b in range(rows_per_tile // 32):
        pltpu.sync_copy(num_sh.at[pl.ds(row0 + b * 32, 32)],
                        msgv.at[pl.ds(0, 32)])
        pltpu.sync_copy(msgv.at[pl.ds(0, 32)],
                        num_out.at[cid, pl.ds(row0 + b * 32, 32)])


def kernel(x, edge_index, W1, a_src1, a_dst1, b1, W2, a_src2, a_dst2, b2,
           lw1, lb1, lw2, lb2):
    n, f_in = x.shape
    e = edge_index.shape[1]
    heads, c = a_src1.shape
    out_dim = lw2.shape[1]

    npad = -(-n // (_NS * _CH1)) * (_NS * _CH1)
    npad = -(-npad // _TCR) * _TCR
    epad = -(-e // (_NW * _CH1)) * (_NW * _CH1)
    ep_w = epad // _NW
    grid = npad // _TCR

    xp = jnp.zeros((npad, f_in), F32).at[:n].set(x)
    pad_node = jnp.int32(n)
    s_pad = jnp.full((epad,), pad_node, jnp.int32).at[:e].set(edge_index[0])
    d_pad = jnp.full((epad,), pad_node, jnp.int32).at[:e].set(edge_index[1])
    w12 = jnp.concatenate([W1, W2], axis=1)
    asf = jnp.concatenate([a_src1.reshape(-1), a_src2.reshape(-1)])[None, :]
    adf = jnp.concatenate([a_dst1.reshape(-1), a_dst2.reshape(-1)])[None, :]
    gi = jnp.arange(_XWW, dtype=jnp.int32)
    s2 = (gi[:, None] // c == jnp.arange(_G)[None, :]).astype(F32)
    oc = jnp.where(gi < heads * c, gi % c, c + gi % c)
    sc2 = (oc[:, None] == jnp.arange(2 * c)[None, :]).astype(F32)
    b12 = jnp.concatenate([b1, b2])[None, :]

    full = lambda shape: pl.BlockSpec(shape, lambda i: (0,) * len(shape))
    rowblk = lambda w: pl.BlockSpec((_TCR, w), lambda i: (i, 0))
    xw_tab, atttab, exself, xself = pl.pallas_call(
        _tc1_body,
        grid=(grid,),
        in_specs=[rowblk(f_in), full((f_in, _XWW)), full((1, _XWW)),
                  full((1, _XWW)), full((_XWW, _G)), full((f_in, 4 * out_dim)),
                  full((1, 4 * out_dim)), full((4 * out_dim, out_dim)),
                  full((1, out_dim))],
        out_specs=[rowblk(_XWW), rowblk(2 * _G), rowblk(_G),
                   rowblk(out_dim)],
        out_shape=[jax.ShapeDtypeStruct((npad, _XWW), F32),
                   jax.ShapeDtypeStruct((npad, 2 * _G), F32),
                   jax.ShapeDtypeStruct((npad, _G), F32),
                   jax.ShapeDtypeStruct((npad, out_dim), F32)],
    )(xp, w12, asf, adf, s2, lw1, lb1[None, :], lw2, lb2[None, :])

    mesh = plsc.VectorSubcoreMesh(core_axis_name="c", subcore_axis_name="s",
                                  num_cores=_NC, num_subcores=_NS)
    sc1 = pl.kernel(
        functools.partial(_sc1_body, npad, ep_w),
        out_type=(jax.ShapeDtypeStruct((epad, _G), F32),
                  jax.ShapeDtypeStruct((_NC, npad // 2, 2 * _G), F32)),
        mesh=mesh,
        scratch_types=[pltpu.VMEM((_CH1,), jnp.int32),
                       pltpu.VMEM((_CH1,), jnp.int32),
                       pltpu.VMEM((_CH1,), jnp.int32),
                       pltpu.VMEM((_CH1, 2 * _G), F32),
                       pltpu.VMEM((_CH1, 2 * _G), F32),
                       pltpu.VMEM((_CH1, _G), F32),
                       pltpu.VMEM((_CH1, 2 * _G), F32),
                       pltpu.VMEM_SHARED((npad // 2, 2 * _G), F32),
                       pltpu.SemaphoreType.DMA,
                       pltpu.SemaphoreType.DMA],
    )
    ex_buf, den_p = sc1(atttab, s_pad, d_pad)
    den_p = den_p.reshape(_NC, npad, _G)

    rden, selfnum = pl.pallas_call(
        _tc2_body,
        grid=(grid,),
        in_specs=[pl.BlockSpec((_NC, _TCR, _G), lambda i: (0, i, 0)),
                  rowblk(_G), rowblk(_XWW), full((_G, _XWW)),
                  full((_XWW, 2 * c))],
        out_specs=[rowblk(2 * _G), rowblk(2 * c)],
        out_shape=[jax.ShapeDtypeStruct((npad, 2 * _G), F32),
                   jax.ShapeDtypeStruct((npad, 2 * c), F32)],
    )(den_p, exself, xw_tab, s2.T, sc2)

    sc2k = pl.kernel(
        functools.partial(_sc2_body, npad, ep_w),
        out_type=jax.ShapeDtypeStruct((_NC, npad // 4, 2 * _G), F32),
        mesh=mesh,
        scratch_types=[pltpu.VMEM((_CH2,), jnp.int32),
                       pltpu.VMEM((_CH2,), jnp.int32),
                       pltpu.VMEM((_CH2,), jnp.int32),
                       pltpu.VMEM((_CH2, _XWW), F32),
                       pltpu.VMEM((_CH2, 2 * _G), F32),
                       pltpu.VMEM((_CH2, _G), F32),
                       pltpu.VMEM((_CH2, 2 * _G), F32),
                       pltpu.VMEM_SHARED((npad // 4, 2 * _G), F32),
                       pltpu.SemaphoreType.DMA,
                       pltpu.SemaphoreType.DMA],
    )
    num_p = sc2k(xw_tab, rden, ex_buf, s_pad, d_pad)
    num_p = num_p.reshape(_NC, npad, 2 * c)

    out12 = pl.pallas_call(
        _tc3_body,
        grid=(grid,),
        in_specs=[pl.BlockSpec((_NC, _TCR, 2 * c), lambda i: (0, i, 0)),
                  rowblk(2 * c), full((1, 2 * c))],
        out_specs=rowblk(2 * c),
        out_shape=jax.ShapeDtypeStruct((npad, 2 * c), F32),
    )(num_p, selfnum, b12)

    return (out12[:n, :c], out12[:n, c:], xself[:n])
